# baseline (speedup 1.0000x reference)
import functools

import jax
import jax.numpy as jnp
from jax import lax
from jax.experimental import pallas as pl
from jax.experimental.pallas import tpu as pltpu
from jax.experimental.pallas import tpu_sc as plsc

_NC = 2
_NS = 16
_C = 128
_G = 64
_H = 128


def _edge_mlp_body(ea_ref, w_ref, b_ref, out_ref):
    out_ref[...] = jnp.dot(ea_ref[...], w_ref[...],
                           preferred_element_type=jnp.float32) + b_ref[...]


def _edge_mlp(ea_packed, w8, b8):
    Ep = ea_packed.shape[0]
    BP = 160
    return pl.pallas_call(
        _edge_mlp_body,
        grid=(Ep // BP,),
        in_specs=[
            pl.BlockSpec((BP, 128), lambda i: (i, 0)),
            pl.BlockSpec((128, 8 * _H), lambda i: (0, 0)),
            pl.BlockSpec((1, 8 * _H), lambda i: (0, 0)),
        ],
        out_specs=pl.BlockSpec((BP, 8 * _H), lambda i: (i, 0)),
        out_shape=jax.ShapeDtypeStruct((Ep, 8 * _H), jnp.float32),
    )(ea_packed, w8, b8)


def _sc_msg(h, e_l, src_g, dst_g, N, E):
    NW = _NC * _NS
    NCH = E // _C
    maxch = -(-NCH // NW)
    shift = NW.bit_length() - 1
    assert 1 << shift == NW
    zrows = 128
    Np = ((N + _NS * zrows - 1) // (_NS * zrows)) * (_NS * zrows)
    rows_per_sub = Np // _NS
    nz = rows_per_sub // zrows
    mesh = plsc.VectorSubcoreMesh(core_axis_name="c", subcore_axis_name="s")

    @functools.partial(
        pl.kernel,
        out_type=jax.ShapeDtypeStruct((_NC, Np, _H), jnp.float32),
        mesh=mesh,
        scratch_types=[
            pltpu.VMEM_SHARED((Np, _H), jnp.float32),
            pltpu.VMEM((4, 1, _C), jnp.int32),
            pltpu.VMEM((4, 1, _C), jnp.int32),
            pltpu.VMEM((2, _C, _H), jnp.float32),
        ] + [pltpu.SemaphoreType.DMA] * 12,
    )
    def k(h_hbm, e_hbm, src_hbm, dst_hbm, out_hbm,
          agg_sh, sbuf, dbuf, ebuf, *sems):
        ssem = sems[0:4]
        dsem = sems[4:8]
        esem = sems[8:10]
        gsem = sems[10:12]
        c = lax.axis_index("c")
        s = lax.axis_index("s")
        wid = c * _NS + s

        def zb(i, carry):
            ebuf[0, i // 8, pl.ds((i % 8) * 16, 16)] = jnp.zeros((16,),
                                                                 jnp.float32)
            return carry
        lax.fori_loop(0, zrows * 8, zb, 0)
        for t in range(nz):
            pltpu.sync_copy(
                ebuf.at[0],
                agg_sh.at[pl.ds(s * rows_per_sub + t * zrows, zrows)])
        plsc.subcore_barrier()

        c0 = lax.shift_right_logical(NCH * wid, shift)
        c1 = lax.shift_right_logical(NCH * (wid + 1), shift)

        def e_rows(i):
            return e_hbm.at[pl.ds(i * _C, _C)]

        def issue_idx(t, i):
            pltpu.async_copy(src_hbm.at[pl.ds(i, 1)], sbuf.at[t], ssem[t])
            pltpu.async_copy(dst_hbm.at[pl.ds(i, 1)], dbuf.at[t], dsem[t])

        def issue_e(p, i):
            pltpu.async_copy(e_rows(i), ebuf.at[p], esem[p])

        for t in range(4):
            issue_idx(t, c0 + t)
        for p in range(2):
            issue_e(p, c0 + p)

        def quad(j, carry):
            base = c0 + 4 * j
            for b in range(4):
                i = base + b
                bp = b & 1

                @pl.when(i < c1)
                def _():
                    pltpu.make_async_copy(e_rows(i), ebuf.at[bp],
                                          esem[bp]).wait()
                    pltpu.make_async_copy(src_hbm.at[pl.ds(i, 1)],
                                          sbuf.at[b], ssem[b]).wait()
                    pltpu.async_copy(h_hbm.at[sbuf.at[b, 0]], ebuf.at[bp],
                                     gsem[bp], add=True)

                ip = i - 1
                pp = 1 - bp
                tp = (b - 1) % 4

                @pl.when((ip >= c0) & (ip < c1))
                def _():
                    pltpu.make_async_copy(h_hbm.at[sbuf.at[tp, 0]],
                                          ebuf.at[pp], gsem[pp]).wait()

                    def relu_row(r, cc):
                        for u in range(2):
                            for q in range(_H // 16):
                                sl = pl.ds(q * 16, 16)
                                ebuf[pp, 2 * r + u, sl] = jnp.maximum(
                                    ebuf[pp, 2 * r + u, sl], 0.0)
                        return cc
                    lax.fori_loop(0, _C // 2, relu_row, 0)
                    pltpu.make_async_copy(dst_hbm.at[pl.ds(ip, 1)],
                                          dbuf.at[tp], dsem[tp]).wait()
                    pltpu.sync_copy(ebuf.at[pp], agg_sh.at[dbuf.at[tp, 0]],
                                    add=True)

                    @pl.when(ip + 2 < c1)
                    def _():
                        issue_e(pp, ip + 2)

                    @pl.when(ip + 4 < c1)
                    def _():
                        issue_idx(tp, ip + 4)
            return carry
        lax.fori_loop(0, (maxch + 4) // 4, quad, 0)
        plsc.subcore_barrier()
        pltpu.sync_copy(agg_sh.at[pl.ds(s * rows_per_sub, rows_per_sub)],
                        out_hbm.at[c, pl.ds(s * rows_per_sub, rows_per_sub)])

    return k(h, e_l, src_g, dst_g)


def _dense_body(h_ref, agg_ref, w1_ref, b1_ref, g_ref, be_ref, w2_ref, b2_ref,
                bt_ref, hn_ref, pool_ref):
    N = h_ref.shape[0]
    z = h_ref[...] + agg_ref[0, :N, :] + agg_ref[1, :N, :]
    z = jnp.dot(z, w1_ref[...], preferred_element_type=jnp.float32) + b1_ref[...]
    m = jnp.mean(z, axis=0, keepdims=True)
    v = jnp.mean((z - m) ** 2, axis=0, keepdims=True)
    z = (z - m) / jnp.sqrt(v + 1e-5) * g_ref[...] + be_ref[...]
    z = jnp.maximum(z, 0.0)
    z = jnp.dot(z, w2_ref[...], preferred_element_type=jnp.float32) + b2_ref[...]
    hn = jnp.maximum(z, 0.0)
    hn_ref[...] = hn
    onehot_t = (lax.broadcasted_iota(jnp.int32, (_G, N), 0)
                == bt_ref[...]).astype(jnp.float32)
    pool_ref[...] = jnp.dot(onehot_t, hn, preferred_element_type=jnp.float32)


def _dense(h, agg2, p, l, bt_row):
    N = h.shape[0]
    return pl.pallas_call(
        _dense_body,
        out_shape=[
            jax.ShapeDtypeStruct((N, _H), jnp.float32),
            jax.ShapeDtypeStruct((_G, _H), jnp.float32),
        ],
    )(h, agg2,
      p['c%d_W1' % l], p['c%d_b1' % l].reshape(1, _H),
      p['c%d_gamma' % l].reshape(1, _H), p['c%d_beta' % l].reshape(1, _H),
      p['c%d_W2' % l], p['c%d_b2' % l].reshape(1, _H),
      bt_row)


def _head_body(pool_ref, w1_ref, b1_ref, w2_ref, b2_ref, out_ref):
    hc = jnp.concatenate([pool_ref[l] for l in range(4)], axis=1)
    y = jnp.dot(hc, w1_ref[...], preferred_element_type=jnp.float32) + b1_ref[...]
    y = jnp.maximum(y, 0.0)
    out_ref[...] = jnp.dot(y, w2_ref[...],
                           preferred_element_type=jnp.float32) + b2_ref[...]


def _head(pools, p):
    return pl.pallas_call(
        _head_body,
        out_shape=jax.ShapeDtypeStruct((_G, 1), jnp.float32),
    )(pools, p['lin1_W'], p['lin1_b'].reshape(1, 4 * _H),
      p['lin2_W'], p['lin2_b'].reshape(1, 1))


def kernel(x, edge_index, edge_attr, batch, params):
    N = x.shape[0]
    E = edge_index.shape[1]
    nchunks = E // _C
    nchp = nchunks + 8
    src_g = jnp.pad(edge_index[0].reshape(nchunks, _C),
                    ((0, nchp - nchunks), (0, 0)))
    dst_g = jnp.pad(edge_index[1].reshape(nchunks, _C),
                    ((0, nchp - nchunks), (0, 0)))
    bt_row = batch.reshape(1, N)

    ea_packed = edge_attr.reshape(E // 8, 128)
    eye8 = jnp.eye(8, dtype=jnp.float32)
    es = []
    for l in range(4):
        w8 = jnp.kron(eye8, params['c%d_We' % l])
        b8 = jnp.tile(params['c%d_be' % l], 8).reshape(1, 8 * _H)
        es.append(_edge_mlp(ea_packed, w8, b8).reshape(E, _H))

    h = x
    pools = []
    for l in range(4):
        agg2 = _sc_msg(h, es[l], src_g, dst_g, N, E)
        h, pool = _dense(h, agg2, params, l, bt_row)
        pools.append(pool)
    return _head(jnp.stack(pools), params)


# speedup vs baseline: 1.0091x; 1.0091x over previous
import functools

import jax
import jax.numpy as jnp
from jax import lax
from jax.experimental import pallas as pl
from jax.experimental.pallas import tpu as pltpu
from jax.experimental.pallas import tpu_sc as plsc

_NC = 2
_NS = 16
_C = 128
_G = 64
_H = 128


def _edge_mlp_body(ea_ref, w_ref, b_ref, out_ref):
    out_ref[...] = jnp.dot(ea_ref[...], w_ref[...],
                           preferred_element_type=jnp.float32) + b_ref[...]


def _edge_mlp(ea_packed, w8, b8):
    Ep = ea_packed.shape[0]
    BP = 160
    return pl.pallas_call(
        _edge_mlp_body,
        grid=(Ep // BP,),
        in_specs=[
            pl.BlockSpec((BP, 128), lambda i: (i, 0)),
            pl.BlockSpec((128, 8 * _H), lambda i: (0, 0)),
            pl.BlockSpec((1, 8 * _H), lambda i: (0, 0)),
        ],
        out_specs=pl.BlockSpec((BP, 8 * _H), lambda i: (i, 0)),
        out_shape=jax.ShapeDtypeStruct((Ep, 8 * _H), jnp.float32),
    )(ea_packed, w8, b8)


def _sc_msg(h, e_l, src_g, dst_g, N, E):
    NW = _NC * _NS
    NCH = E // _C
    maxch = -(-NCH // NW)
    shift = NW.bit_length() - 1
    assert 1 << shift == NW
    zrows = 128
    Np = ((N + _NS * zrows - 1) // (_NS * zrows)) * (_NS * zrows)
    rows_per_sub = Np // _NS
    nz = rows_per_sub // zrows
    mesh = plsc.VectorSubcoreMesh(core_axis_name="c", subcore_axis_name="s")

    @functools.partial(
        pl.kernel,
        out_type=jax.ShapeDtypeStruct((_NC, Np, _H), jnp.float32),
        mesh=mesh,
        scratch_types=[
            pltpu.VMEM_SHARED((Np, _H), jnp.float32),
            pltpu.VMEM((4, 1, _C), jnp.int32),
            pltpu.VMEM((4, 1, _C), jnp.int32),
            pltpu.VMEM((2, _C, _H), jnp.float32),
        ] + [pltpu.SemaphoreType.DMA] * 12,
    )
    def k(h_hbm, e_hbm, src_hbm, dst_hbm, out_hbm,
          agg_sh, sbuf, dbuf, ebuf, *sems):
        ssem = sems[0:4]
        dsem = sems[4:8]
        esem = sems[8:10]
        gsem = sems[10:12]
        c = lax.axis_index("c")
        s = lax.axis_index("s")
        wid = c * _NS + s

        def zb(i, carry):
            ebuf[0, i // 8, pl.ds((i % 8) * 16, 16)] = jnp.zeros((16,),
                                                                 jnp.float32)
            return carry
        lax.fori_loop(0, zrows * 8, zb, 0)
        for t in range(nz):
            pltpu.sync_copy(
                ebuf.at[0],
                agg_sh.at[pl.ds(s * rows_per_sub + t * zrows, zrows)])
        plsc.subcore_barrier()

        c0 = lax.shift_right_logical(NCH * wid, shift)
        c1 = lax.shift_right_logical(NCH * (wid + 1), shift)

        def e_rows(i):
            return e_hbm.at[pl.ds(i * _C, _C)]

        def issue_idx(t, i):
            pltpu.async_copy(src_hbm.at[pl.ds(i, 1)], sbuf.at[t], ssem[t])
            pltpu.async_copy(dst_hbm.at[pl.ds(i, 1)], dbuf.at[t], dsem[t])

        def issue_e(p, i):
            pltpu.async_copy(e_rows(i), ebuf.at[p], esem[p])

        for t in range(4):
            issue_idx(t, c0 + t)
        for p in range(2):
            issue_e(p, c0 + p)

        def quad(j, carry):
            base = c0 + 4 * j
            for b in range(4):
                i = base + b
                bp = b & 1

                @pl.when(i < c1)
                def _():
                    pltpu.make_async_copy(e_rows(i), ebuf.at[bp],
                                          esem[bp]).wait()
                    pltpu.make_async_copy(src_hbm.at[pl.ds(i, 1)],
                                          sbuf.at[b], ssem[b]).wait()
                    pltpu.async_copy(h_hbm.at[sbuf.at[b, 0]], ebuf.at[bp],
                                     gsem[bp], add=True)

                ip = i - 1
                pp = 1 - bp
                tp = (b - 1) % 4

                @pl.when((ip >= c0) & (ip < c1))
                def _():
                    pltpu.make_async_copy(h_hbm.at[sbuf.at[tp, 0]],
                                          ebuf.at[pp], gsem[pp]).wait()

                    def relu_row(r, cc):
                        for u in range(2):
                            for q in range(_H // 16):
                                sl = pl.ds(q * 16, 16)
                                ebuf[pp, 2 * r + u, sl] = jnp.maximum(
                                    ebuf[pp, 2 * r + u, sl], 0.0)
                        return cc
                    lax.fori_loop(0, _C // 2, relu_row, 0)
                    pltpu.make_async_copy(dst_hbm.at[pl.ds(ip, 1)],
                                          dbuf.at[tp], dsem[tp]).wait()
                    pltpu.sync_copy(ebuf.at[pp], agg_sh.at[dbuf.at[tp, 0]],
                                    add=True)

                    @pl.when(ip + 2 < c1)
                    def _():
                        issue_e(pp, ip + 2)

                    @pl.when(ip + 4 < c1)
                    def _():
                        issue_idx(tp, ip + 4)
            return carry
        lax.fori_loop(0, (maxch + 4) // 4, quad, 0)
        plsc.subcore_barrier()
        pltpu.sync_copy(agg_sh.at[pl.ds(s * rows_per_sub, rows_per_sub)],
                        out_hbm.at[c, pl.ds(s * rows_per_sub, rows_per_sub)])

    return k(h, e_l, src_g, dst_g)


def _dense_body(h_ref, agg_ref, w1_ref, b1_ref, g_ref, be_ref, w2_ref, b2_ref,
                bt_ref, hn_ref, pool_ref):
    N = h_ref.shape[0]
    z = h_ref[...] + agg_ref[0, :N, :] + agg_ref[1, :N, :]
    z = jnp.dot(z, w1_ref[...], preferred_element_type=jnp.float32) + b1_ref[...]
    m = jnp.mean(z, axis=0, keepdims=True)
    v = jnp.mean((z - m) ** 2, axis=0, keepdims=True)
    z = (z - m) / jnp.sqrt(v + 1e-5) * g_ref[...] + be_ref[...]
    z = jnp.maximum(z, 0.0)
    z = jnp.dot(z, w2_ref[...], preferred_element_type=jnp.float32) + b2_ref[...]
    hn = jnp.maximum(z, 0.0)
    hn_ref[...] = hn
    onehot_t = (lax.broadcasted_iota(jnp.int32, (_G, N), 0)
                == bt_ref[...]).astype(jnp.float32)
    pool_ref[...] = jnp.dot(onehot_t, hn, preferred_element_type=jnp.float32)


def _dense(h, agg2, p, l, bt_row):
    N = h.shape[0]
    return pl.pallas_call(
        _dense_body,
        out_shape=[
            jax.ShapeDtypeStruct((N, _H), jnp.float32),
            jax.ShapeDtypeStruct((_G, _H), jnp.float32),
        ],
    )(h, agg2,
      p['c%d_W1' % l], p['c%d_b1' % l].reshape(1, _H),
      p['c%d_gamma' % l].reshape(1, _H), p['c%d_beta' % l].reshape(1, _H),
      p['c%d_W2' % l], p['c%d_b2' % l].reshape(1, _H),
      bt_row)


def _head_body(pool_ref, w1_ref, b1_ref, w2_ref, b2_ref, out_ref):
    hc = jnp.concatenate([pool_ref[l] for l in range(4)], axis=1)
    y = jnp.dot(hc, w1_ref[...], preferred_element_type=jnp.float32) + b1_ref[...]
    y = jnp.maximum(y, 0.0)
    out_ref[...] = jnp.dot(y, w2_ref[...],
                           preferred_element_type=jnp.float32) + b2_ref[...]


def _head(pools, p):
    return pl.pallas_call(
        _head_body,
        out_shape=jax.ShapeDtypeStruct((_G, 1), jnp.float32),
    )(pools, p['lin1_W'], p['lin1_b'].reshape(1, 4 * _H),
      p['lin2_W'], p['lin2_b'].reshape(1, 1))


def kernel(x, edge_index, edge_attr, batch, params):
    N = x.shape[0]
    E = edge_index.shape[1]
    nchunks = E // _C
    nchp = nchunks + 8
    src_g = jnp.pad(edge_index[0].reshape(nchunks, _C),
                    ((0, nchp - nchunks), (0, 0)))
    dst_g = jnp.pad(edge_index[1].reshape(nchunks, _C),
                    ((0, nchp - nchunks), (0, 0)))
    bt_row = batch.reshape(1, N)

    ea_packed = edge_attr.reshape(E // 8, 128).astype(jnp.bfloat16)
    eye8 = jnp.eye(8, dtype=jnp.float32)
    es = []
    for l in range(4):
        w8 = jnp.kron(eye8, params['c%d_We' % l]).astype(jnp.bfloat16)
        b8 = jnp.tile(params['c%d_be' % l], 8).reshape(1, 8 * _H)
        es.append(_edge_mlp(ea_packed, w8, b8).reshape(E, _H))

    h = x
    pools = []
    for l in range(4):
        agg2 = _sc_msg(h, es[l], src_g, dst_g, N, E)
        h, pool = _dense(h, agg2, params, l, bt_row)
        pools.append(pool)
    return _head(jnp.stack(pools), params)


# speedup vs baseline: 1.4673x; 1.4541x over previous
import functools

import jax
import jax.numpy as jnp
from jax import lax
from jax.experimental import pallas as pl
from jax.experimental.pallas import tpu as pltpu
from jax.experimental.pallas import tpu_sc as plsc

_NC = 2
_NS = 16
_C = 128
_G = 64
_H = 128


_BP = 160


def _edge_mlp_body(ea_ref, w_ref, b_ref, out_ref):
    prod = jnp.dot(ea_ref[...], w_ref[...],
                   preferred_element_type=jnp.float32) + b_ref[...]
    for k in range(8):
        out_ref[pl.ds(k * _BP, _BP), :] = prod[:, k * _H:(k + 1) * _H]


def _edge_mlp(ea_packed, w8, b8, E):
    Ep = ea_packed.shape[0]
    return pl.pallas_call(
        _edge_mlp_body,
        grid=(Ep // _BP,),
        in_specs=[
            pl.BlockSpec((_BP, 128), lambda i: (i, 0)),
            pl.BlockSpec((128, 8 * _H), lambda i: (0, 0)),
            pl.BlockSpec((1, 8 * _H), lambda i: (0, 0)),
        ],
        out_specs=pl.BlockSpec((8 * _BP, _H), lambda i: (i, 0)),
        out_shape=jax.ShapeDtypeStruct((E, _H), jnp.float32),
    )(ea_packed, w8, b8)


def _sc_msg(h, e_l, src_g, dst_g, N, E):
    NW = _NC * _NS
    NCH = E // _C
    maxch = -(-NCH // NW)
    shift = NW.bit_length() - 1
    assert 1 << shift == NW
    zrows = 128
    Np = ((N + _NS * zrows - 1) // (_NS * zrows)) * (_NS * zrows)
    rows_per_sub = Np // _NS
    nz = rows_per_sub // zrows
    mesh = plsc.VectorSubcoreMesh(core_axis_name="c", subcore_axis_name="s")

    @functools.partial(
        pl.kernel,
        out_type=jax.ShapeDtypeStruct((_NC, Np, _H), jnp.float32),
        mesh=mesh,
        scratch_types=[
            pltpu.VMEM_SHARED((Np, _H), jnp.float32),
            pltpu.VMEM((4, 1, _C), jnp.int32),
            pltpu.VMEM((4, 1, _C), jnp.int32),
            pltpu.VMEM((2, _C, _H), jnp.float32),
        ] + [pltpu.SemaphoreType.DMA] * 12,
    )
    def k(h_hbm, e_hbm, src_hbm, dst_hbm, out_hbm,
          agg_sh, sbuf, dbuf, ebuf, *sems):
        ssem = sems[0:4]
        dsem = sems[4:8]
        esem = sems[8:10]
        gsem = sems[10:12]
        c = lax.axis_index("c")
        s = lax.axis_index("s")
        wid = c * _NS + s

        def zb(i, carry):
            ebuf[0, i // 8, pl.ds((i % 8) * 16, 16)] = jnp.zeros((16,),
                                                                 jnp.float32)
            return carry
        lax.fori_loop(0, zrows * 8, zb, 0)
        for t in range(nz):
            pltpu.sync_copy(
                ebuf.at[0],
                agg_sh.at[pl.ds(s * rows_per_sub + t * zrows, zrows)])
        plsc.subcore_barrier()

        c0 = lax.shift_right_logical(NCH * wid, shift)
        c1 = lax.shift_right_logical(NCH * (wid + 1), shift)

        def e_rows(i):
            return e_hbm.at[pl.ds(i * _C, _C)]

        def issue_idx(t, i):
            pltpu.async_copy(src_hbm.at[pl.ds(i, 1)], sbuf.at[t], ssem[t])
            pltpu.async_copy(dst_hbm.at[pl.ds(i, 1)], dbuf.at[t], dsem[t])

        def issue_e(p, i):
            pltpu.async_copy(e_rows(i), ebuf.at[p], esem[p])

        for t in range(4):
            issue_idx(t, c0 + t)
        for p in range(2):
            issue_e(p, c0 + p)

        def quad(j, carry):
            base = c0 + 4 * j
            for b in range(4):
                i = base + b
                bp = b & 1

                @pl.when(i < c1)
                def _():
                    pltpu.make_async_copy(e_rows(i), ebuf.at[bp],
                                          esem[bp]).wait()
                    pltpu.make_async_copy(src_hbm.at[pl.ds(i, 1)],
                                          sbuf.at[b], ssem[b]).wait()
                    pltpu.async_copy(h_hbm.at[sbuf.at[b, 0]], ebuf.at[bp],
                                     gsem[bp], add=True)

                ip = i - 1
                pp = 1 - bp
                tp = (b - 1) % 4

                @pl.when((ip >= c0) & (ip < c1))
                def _():
                    pltpu.make_async_copy(h_hbm.at[sbuf.at[tp, 0]],
                                          ebuf.at[pp], gsem[pp]).wait()

                    def relu_row(r, cc):
                        for u in range(2):
                            for q in range(_H // 16):
                                sl = pl.ds(q * 16, 16)
                                ebuf[pp, 2 * r + u, sl] = jnp.maximum(
                                    ebuf[pp, 2 * r + u, sl], 0.0)
                        return cc
                    lax.fori_loop(0, _C // 2, relu_row, 0)
                    pltpu.make_async_copy(dst_hbm.at[pl.ds(ip, 1)],
                                          dbuf.at[tp], dsem[tp]).wait()
                    pltpu.sync_copy(ebuf.at[pp], agg_sh.at[dbuf.at[tp, 0]],
                                    add=True)

                    @pl.when(ip + 2 < c1)
                    def _():
                        issue_e(pp, ip + 2)

                    @pl.when(ip + 4 < c1)
                    def _():
                        issue_idx(tp, ip + 4)
            return carry
        lax.fori_loop(0, (maxch + 4) // 4, quad, 0)
        plsc.subcore_barrier()
        pltpu.sync_copy(agg_sh.at[pl.ds(s * rows_per_sub, rows_per_sub)],
                        out_hbm.at[c, pl.ds(s * rows_per_sub, rows_per_sub)])

    return k(h, e_l, src_g, dst_g)


def _dense_body(h_ref, agg_ref, w1_ref, b1_ref, g_ref, be_ref, w2_ref, b2_ref,
                bt_ref, hn_ref, pool_ref):
    N = h_ref.shape[0]
    z = h_ref[...] + agg_ref[0, :N, :] + agg_ref[1, :N, :]
    z = jnp.dot(z, w1_ref[...], preferred_element_type=jnp.float32) + b1_ref[...]
    m = jnp.mean(z, axis=0, keepdims=True)
    v = jnp.mean((z - m) ** 2, axis=0, keepdims=True)
    z = (z - m) / jnp.sqrt(v + 1e-5) * g_ref[...] + be_ref[...]
    z = jnp.maximum(z, 0.0)
    z = jnp.dot(z, w2_ref[...], preferred_element_type=jnp.float32) + b2_ref[...]
    hn = jnp.maximum(z, 0.0)
    hn_ref[...] = hn
    onehot_t = (lax.broadcasted_iota(jnp.int32, (_G, N), 0)
                == bt_ref[...]).astype(jnp.float32)
    pool_ref[...] = jnp.dot(onehot_t, hn, preferred_element_type=jnp.float32)


def _dense(h, agg2, p, l, bt_row):
    N = h.shape[0]
    return pl.pallas_call(
        _dense_body,
        out_shape=[
            jax.ShapeDtypeStruct((N, _H), jnp.float32),
            jax.ShapeDtypeStruct((_G, _H), jnp.float32),
        ],
    )(h, agg2,
      p['c%d_W1' % l], p['c%d_b1' % l].reshape(1, _H),
      p['c%d_gamma' % l].reshape(1, _H), p['c%d_beta' % l].reshape(1, _H),
      p['c%d_W2' % l], p['c%d_b2' % l].reshape(1, _H),
      bt_row)


def _head_body(pool_ref, w1_ref, b1_ref, w2_ref, b2_ref, out_ref):
    hc = jnp.concatenate([pool_ref[l] for l in range(4)], axis=1)
    y = jnp.dot(hc, w1_ref[...], preferred_element_type=jnp.float32) + b1_ref[...]
    y = jnp.maximum(y, 0.0)
    out_ref[...] = jnp.dot(y, w2_ref[...],
                           preferred_element_type=jnp.float32) + b2_ref[...]


def _head(pools, p):
    return pl.pallas_call(
        _head_body,
        out_shape=jax.ShapeDtypeStruct((_G, 1), jnp.float32),
    )(pools, p['lin1_W'], p['lin1_b'].reshape(1, 4 * _H),
      p['lin2_W'], p['lin2_b'].reshape(1, 1))


def kernel(x, edge_index, edge_attr, batch, params):
    N = x.shape[0]
    E = edge_index.shape[1]
    nchunks = E // _C
    nchp = nchunks + 8
    src_g = jnp.pad(edge_index[0].reshape(nchunks, _C),
                    ((0, nchp - nchunks), (0, 0)))
    dst_g = jnp.pad(edge_index[1].reshape(nchunks, _C),
                    ((0, nchp - nchunks), (0, 0)))
    bt_row = batch.reshape(1, N)

    ea_packed = (edge_attr.reshape(E // (8 * _BP), 8, _BP, 16)
                 .transpose(0, 2, 1, 3)
                 .reshape(E // 8, 128).astype(jnp.bfloat16))
    eye8 = jnp.eye(8, dtype=jnp.float32)
    es = []
    for l in range(4):
        w8 = jnp.kron(eye8, params['c%d_We' % l]).astype(jnp.bfloat16)
        b8 = jnp.tile(params['c%d_be' % l], 8).reshape(1, 8 * _H)
        es.append(_edge_mlp(ea_packed, w8, b8, E))

    h = x
    pools = []
    for l in range(4):
        agg2 = _sc_msg(h, es[l], src_g, dst_g, N, E)
        h, pool = _dense(h, agg2, params, l, bt_row)
        pools.append(pool)
    return _head(jnp.stack(pools), params)
